# Initial kernel scaffold; baseline (speedup 1.0000x reference)
import jax
import jax.numpy as jnp
from jax.experimental import pallas as pl


def kernel(x, edge_index, assign_W, assign_b, proj_W, proj_b, m1_W1, m1_b1, m1_W2, m1_b2, lin1_W, m2_W1, m2_b1, m2_W2, m2_b2, lin2_W, fin_W, fin_b):
    raise NotImplementedError("write your pallas kernel here")


# speedup vs baseline: 4.4811x; 4.4811x over previous
import functools

import jax
import jax.numpy as jnp
from jax import lax
from jax.experimental import pallas as pl
from jax.experimental.pallas import tpu as pltpu
from jax.experimental.pallas import tpu_sc as plsc

_KC = 30
_KP = 32
_DP = 128


def _stage1_body(x_ref, aW_ref, ab_ref, pW_ref, pb_ref, S_ref, Z_ref, SS_ref):
    i = pl.program_id(0)
    x = x_ref[...]
    logits = jnp.dot(x, aW_ref[...], preferred_element_type=jnp.float32)
    logits = logits + ab_ref[...]
    kmask = lax.broadcasted_iota(jnp.int32, logits.shape, 1) < _KC
    logits = jnp.where(kmask, logits, -1e30)
    m = jnp.max(logits, axis=1, keepdims=True)
    e = jnp.exp(logits - m)
    S = e / jnp.sum(e, axis=1, keepdims=True)
    S_ref[...] = S
    Sk = S[:, :_KP]
    Xp = jnp.dot(x, pW_ref[...], preferred_element_type=jnp.float32) + pb_ref[...]
    Zp = lax.dot_general(Sk, Xp, (((0,), (0,)), ((), ())),
                         preferred_element_type=jnp.float32)
    SSp = lax.dot_general(Sk, Sk, (((0,), (0,)), ((), ())),
                          preferred_element_type=jnp.float32)

    @pl.when(i == 0)
    def _init():
        Z_ref[...] = Zp
        SS_ref[...] = SSp

    @pl.when(i != 0)
    def _acc():
        Z_ref[...] += Zp
        SS_ref[...] += SSp


def _make_gather(N, E, D):
    info = plsc.get_sparse_core_info()
    nw = info.num_cores * info.num_subcores
    per_w = E // nw
    ch = 200
    while per_w % ch != 0:
        ch //= 2
    n_ch = per_w // ch
    mesh = plsc.VectorSubcoreMesh(core_axis_name="c", subcore_axis_name="s")

    @functools.partial(
        pl.kernel,
        mesh=mesh,
        out_type=(jax.ShapeDtypeStruct((E, D), jnp.float32),
                  jax.ShapeDtypeStruct((E, D), jnp.float32)),
        scratch_types=[
            pltpu.VMEM((ch,), jnp.int32),
            pltpu.VMEM((ch, D), jnp.float32),
            pltpu.SemaphoreType.DMA,
        ],
    )
    def gather_k(table, row_idx, col_idx, out_r, out_c, idx_v, buf_v, sem):
        wid = lax.axis_index("s") * info.num_cores + lax.axis_index("c")
        base = wid * per_w
        for c in range(n_ch):
            off = base + c * ch
            for idx_hbm, out_hbm in ((row_idx, out_r), (col_idx, out_c)):
                pltpu.sync_copy(idx_hbm.at[pl.ds(off, ch)], idx_v)
                pltpu.async_copy(table.at[idx_v], buf_v, sem).wait()
                pltpu.sync_copy(buf_v, out_hbm.at[pl.ds(off, ch)])

    return gather_k


def _stage3_body(sr_ref, sc_ref, Z_ref, SS_ref,
                 m1a_ref, m1b_ref, m1b1_ref, m1W2_ref, m1b2_ref, lin1_ref,
                 m2a_ref, m2b_ref, m2b1_ref, m2W2_ref, m2b2_ref, lin2_ref,
                 finW_ref, finb_ref,
                 out_ref, mc_ref, ol_ref,
                 adj_ref, vol_ref):
    i = pl.program_id(0)
    sr = sr_ref[:, :_KP]
    sc = sc_ref[:, :_KP]
    adj_p = lax.dot_general(sr, sc, (((0,), (0,)), ((), ())),
                            preferred_element_type=jnp.float32)

    @pl.when(i == 0)
    def _init():
        adj_ref[...] = adj_p
        vol_ref[0, 0] = jnp.sum(sr)

    @pl.when(i != 0)
    def _acc():
        adj_ref[...] += adj_p
        vol_ref[0, 0] += jnp.sum(sr)

    @pl.when(i == pl.num_programs(0) - 1)
    def _finale():
        f32 = jnp.float32
        adj = adj_ref[...]
        vol = vol_ref[0, 0]
        rio = lax.broadcasted_iota(jnp.int32, (_KP, _KP), 0)
        cio = lax.broadcasted_iota(jnp.int32, (_KP, _KP), 1)
        eye30 = jnp.where((rio == cio) & (rio < _KC), 1.0, 0.0).astype(f32)
        cut = jnp.sum(adj * eye30)
        mc_ref[...] = jnp.broadcast_to(-cut / (vol + 1e-9), (1, 1))
        d = SS_ref[...] - eye30
        ol_ref[...] = jnp.broadcast_to(jnp.sqrt(jnp.sum(d * d)), (1, 1))

        emask = (adj > 0).astype(f32)
        e2 = _KP * _KP
        eio = lax.broadcasted_iota(jnp.int32, (e2, _KP), 0)
        aio = lax.broadcasted_iota(jnp.int32, (e2, _KP), 1)
        R = (eio // _KP == aio).astype(f32)
        T = (eio % _KP == aio).astype(f32)
        emask_vec = jnp.sum(
            jnp.dot(R, emask, preferred_element_type=f32) * T,
            axis=1, keepdims=True)

        def pgagg(zx, w1a, w1b, b1, w2, b2, lin_w):
            a = jnp.dot(zx, w1a, preferred_element_type=f32)
            b = jnp.dot(zx, w1b, preferred_element_type=f32)
            h = jnp.maximum(
                jnp.dot(R, a, preferred_element_type=f32)
                + jnp.dot(T, b, preferred_element_type=f32) + b1, 0.0)
            s = jnp.dot(h, w2, preferred_element_type=f32) + b2
            mvec = jax.nn.sigmoid(s) * emask_vec
            zc = jnp.dot(T, zx, preferred_element_type=f32)
            msg = zc * mvec
            agg = lax.dot_general(R, msg, (((0,), (0,)), ((), ())),
                                  preferred_element_type=f32)
            norm = lax.dot_general(R, mvec, (((0,), (0,)), ((), ())),
                                   preferred_element_type=f32)
            combined = agg / (norm + 1e-9) + zx
            return jnp.maximum(
                jnp.dot(combined, lin_w, preferred_element_type=f32), 0.0)

        zx = Z_ref[...]
        h1 = pgagg(zx, m1a_ref[...], m1b_ref[...], m1b1_ref[...],
                   m1W2_ref[...], m1b2_ref[...], lin1_ref[...])
        h2 = pgagg(h1, m2a_ref[...], m2b_ref[...], m2b1_ref[...],
                   m2W2_ref[...], m2b2_ref[...], lin2_ref[...])
        outv = jnp.dot(h2, finW_ref[...], preferred_element_type=f32)
        outv = outv + finb_ref[...]
        out_ref[...] = outv[:_KC, :]


def kernel(x, edge_index, assign_W, assign_b, proj_W, proj_b,
           m1_W1, m1_b1, m1_W2, m1_b2, lin1_W,
           m2_W1, m2_b1, m2_W2, m2_b2, lin2_W, fin_W, fin_b):
    f32 = jnp.float32
    N, Din = x.shape
    E = edge_index.shape[1]
    H = lin1_W.shape[1]
    Dout = fin_W.shape[1]
    pad = _KP - _KC

    aW = jnp.pad(assign_W, ((0, 0), (0, _DP - _KC)))
    ab = jnp.pad(assign_b, (0, _DP - _KC)).reshape(1, _DP)
    pb = proj_b.reshape(1, Din)

    bn = 2000
    while N % bn != 0:
        bn //= 2
    grid1 = N // bn
    S_pad, Z_pad, SS = pl.pallas_call(
        _stage1_body,
        grid=(grid1,),
        in_specs=[
            pl.BlockSpec((bn, Din), lambda i: (i, 0)),
            pl.BlockSpec((Din, _DP), lambda i: (0, 0)),
            pl.BlockSpec((1, _DP), lambda i: (0, 0)),
            pl.BlockSpec((Din, Din), lambda i: (0, 0)),
            pl.BlockSpec((1, Din), lambda i: (0, 0)),
        ],
        out_specs=[
            pl.BlockSpec((bn, _DP), lambda i: (i, 0)),
            pl.BlockSpec((_KP, Din), lambda i: (0, 0)),
            pl.BlockSpec((_KP, _KP), lambda i: (0, 0)),
        ],
        out_shape=[
            jax.ShapeDtypeStruct((N, _DP), f32),
            jax.ShapeDtypeStruct((_KP, Din), f32),
            jax.ShapeDtypeStruct((_KP, _KP), f32),
        ],
    )(x, aW, ab, proj_W, pb)

    row = edge_index[0].astype(jnp.int32)
    col = edge_index[1].astype(jnp.int32)
    Srow, Scol = _make_gather(N, E, _DP)(S_pad, row, col)

    be = 8000
    while E % be != 0:
        be //= 2
    grid3 = E // be
    const = lambda i: (0, 0)
    wspecs = [
        pl.BlockSpec((Din, 64), const), pl.BlockSpec((Din, 64), const),
        pl.BlockSpec((1, 64), const), pl.BlockSpec((64, 1), const),
        pl.BlockSpec((1, 1), const), pl.BlockSpec((Din, H), const),
        pl.BlockSpec((H, 64), const), pl.BlockSpec((H, 64), const),
        pl.BlockSpec((1, 64), const), pl.BlockSpec((64, 1), const),
        pl.BlockSpec((1, 1), const), pl.BlockSpec((H, H), const),
        pl.BlockSpec((H, Dout), const), pl.BlockSpec((1, Dout), const),
    ]
    out, mc, ol = pl.pallas_call(
        _stage3_body,
        grid=(grid3,),
        in_specs=[
            pl.BlockSpec((be, _DP), lambda i: (i, 0)),
            pl.BlockSpec((be, _DP), lambda i: (i, 0)),
            pl.BlockSpec((_KP, Din), const),
            pl.BlockSpec((_KP, _KP), const),
        ] + wspecs,
        out_specs=[
            pl.BlockSpec((_KC, Dout), const),
            pl.BlockSpec((1, 1), const),
            pl.BlockSpec((1, 1), const),
        ],
        out_shape=[
            jax.ShapeDtypeStruct((_KC, Dout), f32),
            jax.ShapeDtypeStruct((1, 1), f32),
            jax.ShapeDtypeStruct((1, 1), f32),
        ],
        scratch_shapes=[
            pltpu.VMEM((_KP, _KP), f32),
            pltpu.SMEM((1, 1), f32),
        ],
    )(Srow, Scol, Z_pad, SS,
      m1_W1[:Din], m1_W1[Din:], m1_b1.reshape(1, 64),
      m1_W2, m1_b2.reshape(1, 1), lin1_W,
      m2_W1[:H], m2_W1[H:], m2_b1.reshape(1, 64),
      m2_W2, m2_b2.reshape(1, 1), lin2_W,
      fin_W, fin_b.reshape(1, Dout))

    return (out, mc[0, 0], ol[0, 0], Z_pad[:_KC], S_pad[:, :_KC])


# speedup vs baseline: 5.4729x; 1.2213x over previous
import functools

import jax
import jax.numpy as jnp
from jax import lax
from jax.experimental import pallas as pl
from jax.experimental.pallas import tpu as pltpu
from jax.experimental.pallas import tpu_sc as plsc

_KC = 30
_KP = 32
_DP = 128


def _stage1_body(x_ref, aW_ref, ab_ref, pW_ref, pb_ref, S_ref, Z_ref, SS_ref):
    i = pl.program_id(0)
    x = x_ref[...]
    logits = jnp.dot(x, aW_ref[...], preferred_element_type=jnp.float32)
    logits = logits + ab_ref[...]
    kmask = lax.broadcasted_iota(jnp.int32, logits.shape, 1) < _KC
    logits = jnp.where(kmask, logits, -1e30)
    m = jnp.max(logits, axis=1, keepdims=True)
    e = jnp.exp(logits - m)
    S = e / jnp.sum(e, axis=1, keepdims=True)
    S_ref[...] = S
    Sk = S[:, :_KP]
    Xp = jnp.dot(x, pW_ref[...], preferred_element_type=jnp.float32) + pb_ref[...]
    Zp = lax.dot_general(Sk, Xp, (((0,), (0,)), ((), ())),
                         preferred_element_type=jnp.float32)
    SSp = lax.dot_general(Sk, Sk, (((0,), (0,)), ((), ())),
                          preferred_element_type=jnp.float32)

    @pl.when(i == 0)
    def _init():
        Z_ref[...] = Zp
        SS_ref[...] = SSp

    @pl.when(i != 0)
    def _acc():
        Z_ref[...] += Zp
        SS_ref[...] += SSp


def _make_gather(N, E, D):
    info = plsc.get_sparse_core_info()
    nw = info.num_cores * info.num_subcores
    per_w = E // nw
    ch = 200
    while per_w % ch != 0:
        ch //= 2
    n_ch = per_w // ch
    mesh = plsc.VectorSubcoreMesh(core_axis_name="c", subcore_axis_name="s")

    @functools.partial(
        pl.kernel,
        mesh=mesh,
        out_type=(jax.ShapeDtypeStruct((E, D), jnp.float32),
                  jax.ShapeDtypeStruct((E, D), jnp.float32)),
        scratch_types=[
            pltpu.VMEM((ch,), jnp.int32),
            pltpu.VMEM((ch,), jnp.int32),
            pltpu.VMEM((ch, D), jnp.float32),
            pltpu.VMEM((ch, D), jnp.float32),
            pltpu.SemaphoreType.DMA,
            pltpu.SemaphoreType.DMA,
            pltpu.SemaphoreType.DMA,
            pltpu.SemaphoreType.DMA,
            pltpu.SemaphoreType.DMA,
            pltpu.SemaphoreType.DMA,
        ],
    )
    def gather_k(table, row_idx, col_idx, out_r, out_c,
                 idx0, idx1, buf0, buf1, si0, si1, sg0, sg1, sw0, sw1):
        wid = lax.axis_index("s") * info.num_cores + lax.axis_index("c")
        base = wid * per_w
        idxb, bufb = (idx0, idx1), (buf0, buf1)
        si, sg, sw = (si0, si1), (sg0, sg1), (sw0, sw1)
        jobs = []
        for idx_hbm, out_hbm in ((row_idx, out_r), (col_idx, out_c)):
            for c in range(n_ch):
                jobs.append((idx_hbm, out_hbm, base + c * ch))
        nj = len(jobs)

        def start_idx(j):
            src, _, off = jobs[j]
            return pltpu.async_copy(src.at[pl.ds(off, ch)], idxb[j % 2],
                                    si[j % 2])

        def start_gather(j):
            return pltpu.async_copy(table.at[idxb[j % 2]], bufb[j % 2],
                                    sg[j % 2])

        def start_write(j):
            _, out, off = jobs[j]
            return pltpu.async_copy(bufb[j % 2], out.at[pl.ds(off, ch)],
                                    sw[j % 2])

        hi = {0: start_idx(0)}
        if nj > 1:
            hi[1] = start_idx(1)
        hi[0].wait()
        hg = {0: start_gather(0)}
        hw = {}
        for j in range(nj):
            hg[j].wait()
            hw[j] = start_write(j)
            if j + 2 < nj:
                hi[j + 2] = start_idx(j + 2)
            if j + 1 < nj:
                hi[j + 1].wait()
                if j >= 1:
                    hw[j - 1].wait()
                hg[j + 1] = start_gather(j + 1)
        if nj >= 2:
            hw[nj - 2].wait()
        hw[nj - 1].wait()

    return gather_k


def _stage3_body(sr_ref, sc_ref, Z_ref, SS_ref,
                 m1a_ref, m1b_ref, m1b1_ref, m1W2_ref, m1b2_ref, lin1_ref,
                 m2a_ref, m2b_ref, m2b1_ref, m2W2_ref, m2b2_ref, lin2_ref,
                 finW_ref, finb_ref,
                 out_ref, mc_ref, ol_ref,
                 adj_ref, vol_ref):
    i = pl.program_id(0)
    sr = sr_ref[:, :_KP]
    sc = sc_ref[:, :_KP]
    adj_p = lax.dot_general(sr, sc, (((0,), (0,)), ((), ())),
                            preferred_element_type=jnp.float32)

    @pl.when(i == 0)
    def _init():
        adj_ref[...] = adj_p
        vol_ref[0, 0] = jnp.sum(sr)

    @pl.when(i != 0)
    def _acc():
        adj_ref[...] += adj_p
        vol_ref[0, 0] += jnp.sum(sr)

    @pl.when(i == pl.num_programs(0) - 1)
    def _finale():
        f32 = jnp.float32
        adj = adj_ref[...]
        vol = vol_ref[0, 0]
        rio = lax.broadcasted_iota(jnp.int32, (_KP, _KP), 0)
        cio = lax.broadcasted_iota(jnp.int32, (_KP, _KP), 1)
        eye30 = jnp.where((rio == cio) & (rio < _KC), 1.0, 0.0).astype(f32)
        cut = jnp.sum(adj * eye30)
        mc_ref[...] = jnp.broadcast_to(-cut / (vol + 1e-9), (1, 1))
        d = SS_ref[...] - eye30
        ol_ref[...] = jnp.broadcast_to(jnp.sqrt(jnp.sum(d * d)), (1, 1))

        emask = (adj > 0).astype(f32)
        e2 = _KP * _KP
        eio = lax.broadcasted_iota(jnp.int32, (e2, _KP), 0)
        aio = lax.broadcasted_iota(jnp.int32, (e2, _KP), 1)
        R = (eio // _KP == aio).astype(f32)
        T = (eio % _KP == aio).astype(f32)
        emask_vec = jnp.sum(
            jnp.dot(R, emask, preferred_element_type=f32) * T,
            axis=1, keepdims=True)

        def pgagg(zx, w1a, w1b, b1, w2, b2, lin_w):
            a = jnp.dot(zx, w1a, preferred_element_type=f32)
            b = jnp.dot(zx, w1b, preferred_element_type=f32)
            h = jnp.maximum(
                jnp.dot(R, a, preferred_element_type=f32)
                + jnp.dot(T, b, preferred_element_type=f32) + b1, 0.0)
            s = jnp.dot(h, w2, preferred_element_type=f32) + b2
            mvec = jax.nn.sigmoid(s) * emask_vec
            zc = jnp.dot(T, zx, preferred_element_type=f32)
            msg = zc * mvec
            agg = lax.dot_general(R, msg, (((0,), (0,)), ((), ())),
                                  preferred_element_type=f32)
            norm = lax.dot_general(R, mvec, (((0,), (0,)), ((), ())),
                                   preferred_element_type=f32)
            combined = agg / (norm + 1e-9) + zx
            return jnp.maximum(
                jnp.dot(combined, lin_w, preferred_element_type=f32), 0.0)

        zx = Z_ref[...]
        h1 = pgagg(zx, m1a_ref[...], m1b_ref[...], m1b1_ref[...],
                   m1W2_ref[...], m1b2_ref[...], lin1_ref[...])
        h2 = pgagg(h1, m2a_ref[...], m2b_ref[...], m2b1_ref[...],
                   m2W2_ref[...], m2b2_ref[...], lin2_ref[...])
        outv = jnp.dot(h2, finW_ref[...], preferred_element_type=f32)
        outv = outv + finb_ref[...]
        out_ref[...] = outv[:_KC, :]


def kernel(x, edge_index, assign_W, assign_b, proj_W, proj_b,
           m1_W1, m1_b1, m1_W2, m1_b2, lin1_W,
           m2_W1, m2_b1, m2_W2, m2_b2, lin2_W, fin_W, fin_b):
    f32 = jnp.float32
    N, Din = x.shape
    E = edge_index.shape[1]
    H = lin1_W.shape[1]
    Dout = fin_W.shape[1]
    pad = _KP - _KC

    aW = jnp.pad(assign_W, ((0, 0), (0, _DP - _KC)))
    ab = jnp.pad(assign_b, (0, _DP - _KC)).reshape(1, _DP)
    pb = proj_b.reshape(1, Din)

    bn = 2000
    while N % bn != 0:
        bn //= 2
    grid1 = N // bn
    S_pad, Z_pad, SS = pl.pallas_call(
        _stage1_body,
        grid=(grid1,),
        in_specs=[
            pl.BlockSpec((bn, Din), lambda i: (i, 0)),
            pl.BlockSpec((Din, _DP), lambda i: (0, 0)),
            pl.BlockSpec((1, _DP), lambda i: (0, 0)),
            pl.BlockSpec((Din, Din), lambda i: (0, 0)),
            pl.BlockSpec((1, Din), lambda i: (0, 0)),
        ],
        out_specs=[
            pl.BlockSpec((bn, _DP), lambda i: (i, 0)),
            pl.BlockSpec((_KP, Din), lambda i: (0, 0)),
            pl.BlockSpec((_KP, _KP), lambda i: (0, 0)),
        ],
        out_shape=[
            jax.ShapeDtypeStruct((N, _DP), f32),
            jax.ShapeDtypeStruct((_KP, Din), f32),
            jax.ShapeDtypeStruct((_KP, _KP), f32),
        ],
    )(x, aW, ab, proj_W, pb)

    row = edge_index[0].astype(jnp.int32)
    col = edge_index[1].astype(jnp.int32)
    Srow, Scol = _make_gather(N, E, _DP)(S_pad, row, col)

    be = 8000
    while E % be != 0:
        be //= 2
    grid3 = E // be
    const = lambda i: (0, 0)
    wspecs = [
        pl.BlockSpec((Din, 64), const), pl.BlockSpec((Din, 64), const),
        pl.BlockSpec((1, 64), const), pl.BlockSpec((64, 1), const),
        pl.BlockSpec((1, 1), const), pl.BlockSpec((Din, H), const),
        pl.BlockSpec((H, 64), const), pl.BlockSpec((H, 64), const),
        pl.BlockSpec((1, 64), const), pl.BlockSpec((64, 1), const),
        pl.BlockSpec((1, 1), const), pl.BlockSpec((H, H), const),
        pl.BlockSpec((H, Dout), const), pl.BlockSpec((1, Dout), const),
    ]
    out, mc, ol = pl.pallas_call(
        _stage3_body,
        grid=(grid3,),
        in_specs=[
            pl.BlockSpec((be, _DP), lambda i: (i, 0)),
            pl.BlockSpec((be, _DP), lambda i: (i, 0)),
            pl.BlockSpec((_KP, Din), const),
            pl.BlockSpec((_KP, _KP), const),
        ] + wspecs,
        out_specs=[
            pl.BlockSpec((_KC, Dout), const),
            pl.BlockSpec((1, 1), const),
            pl.BlockSpec((1, 1), const),
        ],
        out_shape=[
            jax.ShapeDtypeStruct((_KC, Dout), f32),
            jax.ShapeDtypeStruct((1, 1), f32),
            jax.ShapeDtypeStruct((1, 1), f32),
        ],
        scratch_shapes=[
            pltpu.VMEM((_KP, _KP), f32),
            pltpu.SMEM((1, 1), f32),
        ],
    )(Srow, Scol, Z_pad, SS,
      m1_W1[:Din], m1_W1[Din:], m1_b1.reshape(1, 64),
      m1_W2, m1_b2.reshape(1, 1), lin1_W,
      m2_W1[:H], m2_W1[H:], m2_b1.reshape(1, 64),
      m2_W2, m2_b2.reshape(1, 1), lin2_W,
      fin_W, fin_b.reshape(1, Dout))

    return (out, mc[0, 0], ol[0, 0], Z_pad[:_KC], S_pad[:, :_KC])
